# baseline (device time: 186744 ns/iter reference)
import jax
import jax.numpy as jnp
from jax import lax
from jax.experimental import pallas as pl
from jax.experimental.pallas import tpu as pltpu

N_DEV = 4
SCALE = 0.08838834764831843


def _fused(x2, wq, wo, k_ext, v_ext):
    m, d = x2.shape
    dh = 128
    b_sz, skv, _ = k_ext.shape
    h_loc = wq.shape[1] // dh
    hw = h_loc * dh
    mc = m // N_DEV

    def body(x_ref, wq_ref, wo_ref, khbm_ref, vhbm_ref, out_ref,
             o_ref, part_ref, comm_ref, kbuf, vbuf, load_sems,
             send_sems, recv_sems, ag_send_sems, ag_recv_sems):
        my = lax.axis_index("i")
        left = lax.rem(my + (N_DEV - 1), N_DEV)
        right = lax.rem(my + 1, N_DEV)
        base = my * hw

        def ridx(idx):
            return lax.rem(idx + 2 * N_DEV, N_DEV)

        def start_load(bidx, slot):
            copies = [
                pltpu.make_async_copy(
                    khbm_ref.at[bidx, :, pl.ds(base, hw)],
                    kbuf.at[slot], load_sems.at[slot]),
                pltpu.make_async_copy(
                    vhbm_ref.at[bidx, :, pl.ds(base, hw)],
                    vbuf.at[slot], load_sems.at[slot]),
            ]
            for c in copies:
                c.start()
            return copies

        loads = [start_load(my, 0), start_load(ridx(my - 1), 1)]

        barrier_sem = pltpu.get_barrier_semaphore()
        for nbr in (left, right):
            pl.semaphore_signal(
                barrier_sem, inc=1,
                device_id=(nbr,), device_id_type=pl.DeviceIdType.MESH,
            )
        pl.semaphore_wait(barrier_sem, 2)

        def compute_batch(bidx, slot, dst):
            qb = jnp.dot(
                x_ref[pl.ds(bidx * mc, mc), :], wq_ref[...],
                preferred_element_type=jnp.float32,
            ).astype(jnp.bfloat16)
            for h in range(h_loc):
                s = lax.dot_general(
                    qb[:, h * dh:(h + 1) * dh].astype(jnp.float32),
                    kbuf[slot, :, h * dh:(h + 1) * dh],
                    (((1,), (1,)), ((), ())),
                    preferred_element_type=jnp.float32,
                )
                e = jnp.exp(s)
                o = jnp.dot(
                    e, vbuf[slot, :, h * dh:(h + 1) * dh],
                    preferred_element_type=jnp.float32,
                ) / jnp.sum(e, axis=1, keepdims=True)
                o_ref[:, h * dh:(h + 1) * dh] = o.astype(jnp.bfloat16)
            dst[...] = jnp.dot(
                o_ref[...], wo_ref[...], preferred_element_type=jnp.float32
            ).astype(jnp.bfloat16)

        for c in loads[0]:
            c.wait()
        compute_batch(my, 0, comm_ref.at[0])
        for s in range(N_DEV - 1):
            rdma = pltpu.make_async_remote_copy(
                src_ref=comm_ref.at[s],
                dst_ref=comm_ref.at[s + 1],
                send_sem=send_sems.at[s],
                recv_sem=recv_sems.at[s],
                device_id=(right,),
                device_id_type=pl.DeviceIdType.MESH,
            )
            rdma.start()
            if s + 2 < N_DEV:
                loads.append(start_load(ridx(my - s - 2), s % 2))
            for c in loads[s + 1]:
                c.wait()
            compute_batch(ridx(my - s - 1), (s + 1) % 2, part_ref.at[s])
            rdma.wait()
            acc = comm_ref[s + 1].astype(jnp.float32) + part_ref[s].astype(
                jnp.float32
            )
            comm_ref[s + 1] = acc.astype(jnp.bfloat16)

        g = lax.rem(my + 1, N_DEV)
        out_ref[pl.ds(g * mc, mc), :] = comm_ref[N_DEV - 1].astype(jnp.float32)

        ag_rdmas = []
        for j in range(1, N_DEV):
            tgt = lax.rem(my + j, N_DEV)
            m_slot = N_DEV - 1 - j
            rdma = pltpu.make_async_remote_copy(
                src_ref=comm_ref.at[N_DEV - 1],
                dst_ref=comm_ref.at[N_DEV + m_slot],
                send_sem=ag_send_sems.at[j - 1],
                recv_sem=ag_recv_sems.at[m_slot],
                device_id=(tgt,),
                device_id_type=pl.DeviceIdType.MESH,
            )
            rdma.start()
            ag_rdmas.append(rdma)

        for m_slot in (0, 2, 1):
            recv = pltpu.make_async_remote_copy(
                src_ref=comm_ref.at[N_DEV - 1],
                dst_ref=comm_ref.at[N_DEV + m_slot],
                send_sem=ag_send_sems.at[0],
                recv_sem=ag_recv_sems.at[m_slot],
                device_id=(my,),
                device_id_type=pl.DeviceIdType.MESH,
            )
            recv.wait_recv()
            origin = lax.rem(my + m_slot + 2, N_DEV)
            out_ref[pl.ds(origin * mc, mc), :] = comm_ref[
                N_DEV + m_slot
            ].astype(jnp.float32)

        for rdma in ag_rdmas:
            rdma.wait_send()

    return pl.pallas_call(
        body,
        out_shape=jax.ShapeDtypeStruct((m, d), jnp.float32),
        in_specs=[
            pl.BlockSpec(memory_space=pltpu.VMEM),
            pl.BlockSpec(memory_space=pltpu.VMEM),
            pl.BlockSpec(memory_space=pltpu.VMEM),
            pl.BlockSpec(memory_space=pltpu.MemorySpace.HBM),
            pl.BlockSpec(memory_space=pltpu.MemorySpace.HBM),
        ],
        out_specs=pl.BlockSpec(memory_space=pltpu.VMEM),
        scratch_shapes=[
            pltpu.VMEM((mc, h_loc * dh), jnp.bfloat16),
            pltpu.VMEM((N_DEV - 1, mc, d), jnp.bfloat16),
            pltpu.VMEM((2 * N_DEV - 1, mc, d), jnp.bfloat16),
            pltpu.VMEM((2, skv, hw), jnp.float32),
            pltpu.VMEM((2, skv, hw), jnp.float32),
            pltpu.SemaphoreType.DMA((2,)),
            pltpu.SemaphoreType.DMA((N_DEV - 1,)),
            pltpu.SemaphoreType.DMA((N_DEV - 1,)),
            pltpu.SemaphoreType.DMA((N_DEV - 1,)),
            pltpu.SemaphoreType.DMA((N_DEV - 1,)),
        ],
        compiler_params=pltpu.CompilerParams(collective_id=0),
    )(x2, wq, wo, k_ext, v_ext)


def kernel(x, Wq, Wo, K_ext, V_ext):
    b, sq, d = x.shape

    x2 = x.reshape(b * sq, d).astype(jnp.bfloat16)
    wq = (Wq * SCALE).astype(jnp.bfloat16)
    wo = Wo.astype(jnp.bfloat16)

    skv = K_ext.shape[1]
    out = _fused(
        x2, wq, wo,
        K_ext.reshape(b, skv, -1),
        V_ext.reshape(b, skv, -1),
    )
    return out.reshape(b, sq, d)


# device time: 52878 ns/iter; 3.5316x vs baseline; 3.5316x over previous
import jax
import jax.numpy as jnp
from jax import lax
from jax.experimental import pallas as pl
from jax.experimental.pallas import tpu as pltpu

N_DEV = 4
SCALE = 0.08838834764831843


def _fused(x2, wq, wo, k_ext, v_ext):
    m, d = x2.shape
    b_sz, skv, h_tot, dh = k_ext.shape
    h_loc = wq.shape[1] // dh
    mc = m // N_DEV

    def body(x_ref, wq_ref, wo_ref, khbm_ref, vhbm_ref, out_ref,
             o_ref, part_ref, comm_ref, kbuf, vbuf, load_sems,
             send_sems, recv_sems, ag_send_sems, ag_recv_sems):
        my = lax.axis_index("i")
        left = lax.rem(my + (N_DEV - 1), N_DEV)
        right = lax.rem(my + 1, N_DEV)
        base = my * h_loc

        def ridx(idx):
            return lax.rem(idx + 2 * N_DEV, N_DEV)

        def start_load(bidx, slot):
            copies = []
            for h in range(h_loc):
                copies.append(pltpu.make_async_copy(
                    khbm_ref.at[bidx, :, base + h, :],
                    kbuf.at[slot, h], load_sems.at[slot]))
                copies.append(pltpu.make_async_copy(
                    vhbm_ref.at[bidx, :, base + h, :],
                    vbuf.at[slot, h], load_sems.at[slot]))
            for c in copies:
                c.start()
            return copies

        loads = [start_load(my, 0), start_load(ridx(my - 1), 1)]

        barrier_sem = pltpu.get_barrier_semaphore()
        for nbr in (left, right):
            pl.semaphore_signal(
                barrier_sem, inc=1,
                device_id=(nbr,), device_id_type=pl.DeviceIdType.MESH,
            )
        pl.semaphore_wait(barrier_sem, 2)

        def compute_batch(bidx, slot, dst):
            qb = jnp.dot(
                x_ref[pl.ds(bidx * mc, mc), :], wq_ref[...],
                preferred_element_type=jnp.float32,
            ).astype(jnp.bfloat16)
            for h in range(h_loc):
                s = lax.dot_general(
                    qb[:, h * dh:(h + 1) * dh].astype(jnp.float32),
                    kbuf[slot, h],
                    (((1,), (1,)), ((), ())),
                    preferred_element_type=jnp.float32,
                )
                e = jnp.exp(s)
                o = jnp.dot(
                    e, vbuf[slot, h], preferred_element_type=jnp.float32,
                ) / jnp.sum(e, axis=1, keepdims=True)
                o_ref[:, h * dh:(h + 1) * dh] = o.astype(jnp.bfloat16)
            dst[...] = jnp.dot(
                o_ref[...], wo_ref[...], preferred_element_type=jnp.float32
            ).astype(jnp.bfloat16)

        for c in loads[0]:
            c.wait()
        compute_batch(my, 0, comm_ref.at[0])
        for s in range(N_DEV - 1):
            rdma = pltpu.make_async_remote_copy(
                src_ref=comm_ref.at[s],
                dst_ref=comm_ref.at[s + 1],
                send_sem=send_sems.at[s],
                recv_sem=recv_sems.at[s],
                device_id=(right,),
                device_id_type=pl.DeviceIdType.MESH,
            )
            rdma.start()
            if s + 2 < N_DEV:
                loads.append(start_load(ridx(my - s - 2), s % 2))
            for c in loads[s + 1]:
                c.wait()
            compute_batch(ridx(my - s - 1), (s + 1) % 2, part_ref.at[s])
            rdma.wait()
            acc = comm_ref[s + 1].astype(jnp.float32) + part_ref[s].astype(
                jnp.float32
            )
            comm_ref[s + 1] = acc.astype(jnp.bfloat16)

        g = lax.rem(my + 1, N_DEV)
        out_ref[pl.ds(g * mc, mc), :] = comm_ref[N_DEV - 1].astype(jnp.float32)

        ag_rdmas = []
        for j in range(1, N_DEV):
            tgt = lax.rem(my + j, N_DEV)
            m_slot = N_DEV - 1 - j
            rdma = pltpu.make_async_remote_copy(
                src_ref=comm_ref.at[N_DEV - 1],
                dst_ref=comm_ref.at[N_DEV + m_slot],
                send_sem=ag_send_sems.at[j - 1],
                recv_sem=ag_recv_sems.at[m_slot],
                device_id=(tgt,),
                device_id_type=pl.DeviceIdType.MESH,
            )
            rdma.start()
            ag_rdmas.append(rdma)

        for m_slot in (0, 2, 1):
            recv = pltpu.make_async_remote_copy(
                src_ref=comm_ref.at[N_DEV - 1],
                dst_ref=comm_ref.at[N_DEV + m_slot],
                send_sem=ag_send_sems.at[0],
                recv_sem=ag_recv_sems.at[m_slot],
                device_id=(my,),
                device_id_type=pl.DeviceIdType.MESH,
            )
            recv.wait_recv()
            origin = lax.rem(my + m_slot + 2, N_DEV)
            out_ref[pl.ds(origin * mc, mc), :] = comm_ref[
                N_DEV + m_slot
            ].astype(jnp.float32)

        for rdma in ag_rdmas:
            rdma.wait_send()

    return pl.pallas_call(
        body,
        out_shape=jax.ShapeDtypeStruct((m, d), jnp.float32),
        in_specs=[
            pl.BlockSpec(memory_space=pltpu.VMEM),
            pl.BlockSpec(memory_space=pltpu.VMEM),
            pl.BlockSpec(memory_space=pltpu.VMEM),
            pl.BlockSpec(memory_space=pltpu.MemorySpace.HBM),
            pl.BlockSpec(memory_space=pltpu.MemorySpace.HBM),
        ],
        out_specs=pl.BlockSpec(memory_space=pltpu.VMEM),
        scratch_shapes=[
            pltpu.VMEM((mc, h_loc * dh), jnp.bfloat16),
            pltpu.VMEM((N_DEV - 1, mc, d), jnp.bfloat16),
            pltpu.VMEM((2 * N_DEV - 1, mc, d), jnp.bfloat16),
            pltpu.VMEM((2, h_loc, skv, dh), jnp.float32),
            pltpu.VMEM((2, h_loc, skv, dh), jnp.float32),
            pltpu.SemaphoreType.DMA((2,)),
            pltpu.SemaphoreType.DMA((N_DEV - 1,)),
            pltpu.SemaphoreType.DMA((N_DEV - 1,)),
            pltpu.SemaphoreType.DMA((N_DEV - 1,)),
            pltpu.SemaphoreType.DMA((N_DEV - 1,)),
        ],
        compiler_params=pltpu.CompilerParams(collective_id=0),
    )(x2, wq, wo, k_ext, v_ext)


def kernel(x, Wq, Wo, K_ext, V_ext):
    b, sq, d = x.shape

    x2 = x.reshape(b * sq, d).astype(jnp.bfloat16)
    wq = (Wq * SCALE).astype(jnp.bfloat16)
    wo = Wo.astype(jnp.bfloat16)

    out = _fused(x2, wq, wo, K_ext, V_ext)
    return out.reshape(b, sq, d)
